# Initial kernel scaffold; baseline (speedup 1.0000x reference)
import jax
import jax.numpy as jnp
from jax.experimental import pallas as pl


def kernel(pos, node_scalar, W1, b1, W2, b2, masses_table, batch, atomic_numbers):
    raise NotImplementedError("write your pallas kernel here")


# speedup vs baseline: 8.2716x; 8.2716x over previous
import functools

import jax
import jax.numpy as jnp
from jax import lax
from jax.experimental import pallas as pl
from jax.experimental.pallas import tpu as pltpu
from jax.experimental.pallas import tpu_sc as plsc

N = 100000
B = 4096
NODE_DIM = 128
HIDDEN_DIM = 64
NSPECIES = 119

NW = 16
CHUNK = 6272
NCH = 49
NP = NW * CHUNK
LANES = 16
NSEG_W = B // NW


@functools.cache
def _mesh():
    return plsc.VectorSubcoreMesh(
        core_axis_name="c", subcore_axis_name="s", num_cores=1,
        num_subcores=NW)


_SC_PARAMS = pltpu.CompilerParams(needs_layout_passes=False)


def _centroid_body(pos3, an, batch2d, table,
                   cx, cy, cz,
                   pv, rx, ry, rz, rm, anv, idxv, tablev, stripe,
                   accx, accy, accz, accm, sem):
    wid = lax.axis_index("s")
    base = wid * CHUNK

    pltpu.sync_copy(pos3.at[pl.ds(base * 3, CHUNK * 3)], pv)
    pltpu.sync_copy(an.at[pl.ds(base, CHUNK)], anv)
    pltpu.sync_copy(batch2d.at[wid], idxv)
    pltpu.sync_copy(table, tablev)

    for k in range(NSEG_W // LANES):
        stripe[pl.ds(k * LANES, LANES)] = jnp.zeros((LANES,), jnp.float32)
    sbase = wid * NSEG_W
    pltpu.sync_copy(stripe, accx.at[pl.ds(sbase, NSEG_W)])
    pltpu.sync_copy(stripe, accy.at[pl.ds(sbase, NSEG_W)])
    pltpu.sync_copy(stripe, accz.at[pl.ds(sbase, NSEG_W)])
    pltpu.sync_copy(stripe, accm.at[pl.ds(sbase, NSEG_W)])
    plsc.subcore_barrier()

    lane = lax.iota(jnp.int32, LANES)

    def chunk_build(j, carry):
        for k in range(8):
            i = j * 8 + k
            sl = pl.ds(i * LANES, LANES)
            ids = i * LANES + lane
            idx3 = ids * 3
            m16 = plsc.load_gather(tablev, [anv[sl]])
            rm[sl] = m16
            rx[sl] = plsc.load_gather(pv, [idx3]) * m16
            ry[sl] = plsc.load_gather(pv, [idx3 + 1]) * m16
            rz[sl] = plsc.load_gather(pv, [idx3 + 2]) * m16
        rows = pl.ds(j * 128, 128)
        idx = idxv.at[j]
        pltpu.async_copy(rx.at[rows], accx.at[idx], sem, add=True)
        pltpu.async_copy(ry.at[rows], accy.at[idx], sem, add=True)
        pltpu.async_copy(rz.at[rows], accz.at[idx], sem, add=True)
        pltpu.async_copy(rm.at[rows], accm.at[idx], sem, add=True)
        return carry

    def chunk_drain(j, carry):
        rows = pl.ds(j * 128, 128)
        idx = idxv.at[j]
        pltpu.make_async_copy(rx.at[rows], accx.at[idx], sem).wait()
        pltpu.make_async_copy(ry.at[rows], accy.at[idx], sem).wait()
        pltpu.make_async_copy(rz.at[rows], accz.at[idx], sem).wait()
        pltpu.make_async_copy(rm.at[rows], accm.at[idx], sem).wait()
        return carry

    lax.fori_loop(0, NCH, chunk_build, 0)
    lax.fori_loop(0, NCH, chunk_drain, 0)
    plsc.subcore_barrier()

    num_x = rx.at[pl.ds(0, NSEG_W)]
    num_y = ry.at[pl.ds(0, NSEG_W)]
    num_z = rz.at[pl.ds(0, NSEG_W)]
    den = rm.at[pl.ds(0, NSEG_W)]
    pltpu.sync_copy(accx.at[pl.ds(sbase, NSEG_W)], num_x)
    pltpu.sync_copy(accy.at[pl.ds(sbase, NSEG_W)], num_y)
    pltpu.sync_copy(accz.at[pl.ds(sbase, NSEG_W)], num_z)
    pltpu.sync_copy(accm.at[pl.ds(sbase, NSEG_W)], den)

    def divide(k, carry):
        sl = pl.ds(k * LANES, LANES)
        inv = 1.0 / den[sl]
        num_x[sl] = num_x[sl] * inv
        num_y[sl] = num_y[sl] * inv
        num_z[sl] = num_z[sl] * inv
        return carry

    lax.fori_loop(0, NSEG_W // LANES, divide, 0)
    pltpu.sync_copy(num_x, cx.at[pl.ds(sbase, NSEG_W)])
    pltpu.sync_copy(num_y, cy.at[pl.ds(sbase, NSEG_W)])
    pltpu.sync_copy(num_z, cz.at[pl.ds(sbase, NSEG_W)])


@functools.cache
def _centroid_kernel(interpret: bool = False):
    return pl.kernel(
        _centroid_body,
        out_type=[jax.ShapeDtypeStruct((B,), jnp.float32)] * 3,
        mesh=_mesh(),
        scratch_types=[
            pltpu.VMEM((CHUNK * 3,), jnp.float32),
            pltpu.VMEM((CHUNK,), jnp.float32),
            pltpu.VMEM((CHUNK,), jnp.float32),
            pltpu.VMEM((CHUNK,), jnp.float32),
            pltpu.VMEM((CHUNK,), jnp.float32),
            pltpu.VMEM((CHUNK,), jnp.int32),
            pltpu.VMEM((NCH, 128), jnp.int32),
            pltpu.VMEM((128,), jnp.float32),
            pltpu.VMEM((NSEG_W,), jnp.float32),
            pltpu.VMEM_SHARED((B,), jnp.float32),
            pltpu.VMEM_SHARED((B,), jnp.float32),
            pltpu.VMEM_SHARED((B,), jnp.float32),
            pltpu.VMEM_SHARED((B,), jnp.float32),
            pltpu.SemaphoreType.DMA,
        ],
        compiler_params=_SC_PARAMS,
        interpret=interpret,
    )


def _extent_body(pos3, so, batch2d, cxh, cyh, czh,
                 out,
                 pv, sov, idxv, cxv, cyv, czv, contrib, stripe, acc, sem):
    wid = lax.axis_index("s")
    base = wid * CHUNK

    pltpu.sync_copy(pos3.at[pl.ds(base * 3, CHUNK * 3)], pv)
    pltpu.sync_copy(so.at[pl.ds(base, CHUNK)], sov)
    pltpu.sync_copy(batch2d.at[wid], idxv)
    pltpu.sync_copy(cxh, cxv)
    pltpu.sync_copy(cyh, cyv)
    pltpu.sync_copy(czh, czv)

    nseg = NSEG_W
    sbase = wid * nseg
    for k in range(nseg // LANES):
        stripe[pl.ds(k * LANES, LANES)] = jnp.zeros((LANES,), jnp.float32)
    pltpu.sync_copy(stripe, acc.at[pl.ds(sbase, nseg)])
    plsc.subcore_barrier()

    lane = lax.iota(jnp.int32, LANES)

    def chunk_build(j, carry):
        for k in range(8):
            i = j * 8 + k
            sl = pl.ds(i * LANES, LANES)
            ids = i * LANES + lane
            idx3 = ids * 3
            b16 = idxv[j, pl.ds(k * LANES, LANES)]
            dx = plsc.load_gather(pv, [idx3]) - plsc.load_gather(cxv, [b16])
            dy = plsc.load_gather(pv, [idx3 + 1]) - plsc.load_gather(cyv, [b16])
            dz = plsc.load_gather(pv, [idx3 + 2]) - plsc.load_gather(czv, [b16])
            sp = dx * dx + dy * dy + dz * dz
            valid = (base + ids) < N
            contrib[sl] = jnp.where(valid, sov[sl] * sp, 0.0)
        pltpu.async_copy(contrib.at[pl.ds(j * 128, 128)], acc.at[idxv.at[j]],
                         sem, add=True)
        return carry

    def chunk_drain(j, carry):
        pltpu.make_async_copy(contrib.at[pl.ds(j * 128, 128)],
                              acc.at[idxv.at[j]], sem).wait()
        return carry

    lax.fori_loop(0, NCH, chunk_build, 0)
    lax.fori_loop(0, NCH, chunk_drain, 0)
    plsc.subcore_barrier()
    pltpu.sync_copy(acc.at[pl.ds(sbase, nseg)], out.at[pl.ds(sbase, nseg)])


@functools.cache
def _extent_kernel(interpret: bool = False):
    return pl.kernel(
        _extent_body,
        out_type=jax.ShapeDtypeStruct((B,), jnp.float32),
        mesh=_mesh(),
        scratch_types=[
            pltpu.VMEM((CHUNK * 3,), jnp.float32),
            pltpu.VMEM((CHUNK,), jnp.float32),
            pltpu.VMEM((NCH, 128), jnp.int32),
            pltpu.VMEM((B,), jnp.float32),
            pltpu.VMEM((B,), jnp.float32),
            pltpu.VMEM((B,), jnp.float32),
            pltpu.VMEM((CHUNK,), jnp.float32),
            pltpu.VMEM((NSEG_W,), jnp.float32),
            pltpu.VMEM_SHARED((B,), jnp.float32),
            pltpu.SemaphoreType.DMA,
        ],
        compiler_params=_SC_PARAMS,
        interpret=interpret,
    )


def _mlp_body(x_ref, w1_ref, b1_ref, w2t_ref, b2_ref, o_ref):
    h = jnp.dot(x_ref[...], w1_ref[...], preferred_element_type=jnp.float32)
    h = h + b1_ref[...]
    h = h * jax.nn.sigmoid(h)
    s = lax.dot_general(w2t_ref[...], h, (((1,), (1,)), ((), ())),
                        preferred_element_type=jnp.float32)
    o_ref[...] = (s + b2_ref[...])[None]


_MLP_BLOCK = 2048
_MLP_GRID = NP // _MLP_BLOCK


def _mlp(node_scalar, W1, b1, W2, b2):
    return pl.pallas_call(
        _mlp_body,
        grid=(_MLP_GRID,),
        in_specs=[
            pl.BlockSpec((_MLP_BLOCK, NODE_DIM), lambda i: (i, 0)),
            pl.BlockSpec((NODE_DIM, HIDDEN_DIM), lambda i: (0, 0)),
            pl.BlockSpec((HIDDEN_DIM,), lambda i: (0,)),
            pl.BlockSpec((1, HIDDEN_DIM), lambda i: (0, 0)),
            pl.BlockSpec((1,), lambda i: (0,)),
        ],
        out_specs=pl.BlockSpec((1, 1, _MLP_BLOCK), lambda i: (i, 0, 0)),
        out_shape=jax.ShapeDtypeStruct((_MLP_GRID, 1, _MLP_BLOCK),
                                       jnp.float32),
    )(node_scalar, W1, b1, W2.reshape(1, HIDDEN_DIM), b2)


def kernel(pos, node_scalar, W1, b1, W2, b2, masses_table, batch,
           atomic_numbers):
    pad = NP - N
    batch32 = jnp.concatenate(
        [batch.astype(jnp.int32), jnp.zeros((pad,), jnp.int32)])
    an32 = jnp.concatenate(
        [atomic_numbers.astype(jnp.int32),
         jnp.full((pad,), NSPECIES, jnp.int32)])
    table = jnp.concatenate(
        [masses_table, jnp.zeros((128 - NSPECIES,), jnp.float32)])
    pos3 = jnp.concatenate(
        [pos, jnp.zeros((pad, 3), jnp.float32)]).reshape(NP * 3)
    batch2d = batch32.reshape(NW, NCH, 128)

    cx, cy, cz = _centroid_kernel()(pos3, an32, batch2d, table)
    so = _mlp(node_scalar, W1, b1, W2, b2).reshape(NP)
    out = _extent_kernel()(pos3, so, batch2d, cx, cy, cz)
    return out.reshape(B, 1)


# speedup vs baseline: 16.3733x; 1.9795x over previous
import functools

import jax
import jax.numpy as jnp
from jax import lax
from jax.experimental import pallas as pl
from jax.experimental.pallas import tpu as pltpu
from jax.experimental.pallas import tpu_sc as plsc

N = 100000
B = 4096
NODE_DIM = 128
HIDDEN_DIM = 64
NSPECIES = 119

NW = 16
CHUNK = 6272
NCH = 49
NP = NW * CHUNK
LANES = 16
NSEG_W = B // NW


@functools.cache
def _mesh():
    return plsc.VectorSubcoreMesh(
        core_axis_name="c", subcore_axis_name="s", num_cores=1,
        num_subcores=NW)


_SC_PARAMS = pltpu.CompilerParams(needs_layout_passes=False)


def _centroid_body(xs, ys, zs, an, batch2d, table,
                   cx, cy, cz,
                   rx, ry, rz, rm, anv, idxv, tablev, stripe,
                   accx, accy, accz, accm, sem):
    wid = lax.axis_index("s")
    base = wid * CHUNK

    pltpu.sync_copy(xs.at[pl.ds(base, CHUNK)], rx)
    pltpu.sync_copy(ys.at[pl.ds(base, CHUNK)], ry)
    pltpu.sync_copy(zs.at[pl.ds(base, CHUNK)], rz)
    pltpu.sync_copy(an.at[pl.ds(base, CHUNK)], anv)
    pltpu.sync_copy(batch2d.at[wid], idxv)
    pltpu.sync_copy(table, tablev)

    for k in range(NSEG_W // LANES):
        stripe[pl.ds(k * LANES, LANES)] = jnp.zeros((LANES,), jnp.float32)
    sbase = wid * NSEG_W
    pltpu.sync_copy(stripe, accx.at[pl.ds(sbase, NSEG_W)])
    pltpu.sync_copy(stripe, accy.at[pl.ds(sbase, NSEG_W)])
    pltpu.sync_copy(stripe, accz.at[pl.ds(sbase, NSEG_W)])
    pltpu.sync_copy(stripe, accm.at[pl.ds(sbase, NSEG_W)])
    plsc.subcore_barrier()

    def chunk_build(j, carry):
        for k in range(8):
            i = j * 8 + k
            sl = pl.ds(i * LANES, LANES)
            m16 = plsc.load_gather(tablev, [anv[sl]])
            rm[sl] = m16
            rx[sl] = rx[sl] * m16
            ry[sl] = ry[sl] * m16
            rz[sl] = rz[sl] * m16
        rows = pl.ds(j * 128, 128)
        idx = idxv.at[j]
        pltpu.async_copy(rx.at[rows], accx.at[idx], sem, add=True)
        pltpu.async_copy(ry.at[rows], accy.at[idx], sem, add=True)
        pltpu.async_copy(rz.at[rows], accz.at[idx], sem, add=True)
        pltpu.async_copy(rm.at[rows], accm.at[idx], sem, add=True)
        return carry

    def chunk_drain(j, carry):
        rows = pl.ds(j * 128, 128)
        idx = idxv.at[j]
        pltpu.make_async_copy(rx.at[rows], accx.at[idx], sem).wait()
        pltpu.make_async_copy(ry.at[rows], accy.at[idx], sem).wait()
        pltpu.make_async_copy(rz.at[rows], accz.at[idx], sem).wait()
        pltpu.make_async_copy(rm.at[rows], accm.at[idx], sem).wait()
        return carry

    lax.fori_loop(0, NCH, chunk_build, 0)
    lax.fori_loop(0, NCH, chunk_drain, 0)
    plsc.subcore_barrier()

    num_x = rx.at[pl.ds(0, NSEG_W)]
    num_y = ry.at[pl.ds(0, NSEG_W)]
    num_z = rz.at[pl.ds(0, NSEG_W)]
    den = rm.at[pl.ds(0, NSEG_W)]
    pltpu.sync_copy(accx.at[pl.ds(sbase, NSEG_W)], num_x)
    pltpu.sync_copy(accy.at[pl.ds(sbase, NSEG_W)], num_y)
    pltpu.sync_copy(accz.at[pl.ds(sbase, NSEG_W)], num_z)
    pltpu.sync_copy(accm.at[pl.ds(sbase, NSEG_W)], den)

    def divide(k, carry):
        sl = pl.ds(k * LANES, LANES)
        inv = 1.0 / den[sl]
        num_x[sl] = num_x[sl] * inv
        num_y[sl] = num_y[sl] * inv
        num_z[sl] = num_z[sl] * inv
        return carry

    lax.fori_loop(0, NSEG_W // LANES, divide, 0)
    pltpu.sync_copy(num_x, cx.at[pl.ds(sbase, NSEG_W)])
    pltpu.sync_copy(num_y, cy.at[pl.ds(sbase, NSEG_W)])
    pltpu.sync_copy(num_z, cz.at[pl.ds(sbase, NSEG_W)])


@functools.cache
def _centroid_kernel(interpret: bool = False):
    return pl.kernel(
        _centroid_body,
        out_type=[jax.ShapeDtypeStruct((B,), jnp.float32)] * 3,
        mesh=_mesh(),
        scratch_types=[
            pltpu.VMEM((CHUNK,), jnp.float32),
            pltpu.VMEM((CHUNK,), jnp.float32),
            pltpu.VMEM((CHUNK,), jnp.float32),
            pltpu.VMEM((CHUNK,), jnp.float32),
            pltpu.VMEM((CHUNK,), jnp.int32),
            pltpu.VMEM((NCH, 128), jnp.int32),
            pltpu.VMEM((128,), jnp.float32),
            pltpu.VMEM((NSEG_W,), jnp.float32),
            pltpu.VMEM_SHARED((B,), jnp.float32),
            pltpu.VMEM_SHARED((B,), jnp.float32),
            pltpu.VMEM_SHARED((B,), jnp.float32),
            pltpu.VMEM_SHARED((B,), jnp.float32),
            pltpu.SemaphoreType.DMA,
        ],
        compiler_params=_SC_PARAMS,
        interpret=interpret,
    )


def _extent_body(xs, ys, zs, so, batch2d, cxh, cyh, czh,
                 out,
                 xv, yv, zv, sov, idxv, cxv, cyv, czv, contrib, stripe, acc,
                 sem):
    wid = lax.axis_index("s")
    base = wid * CHUNK

    pltpu.sync_copy(xs.at[pl.ds(base, CHUNK)], xv)
    pltpu.sync_copy(ys.at[pl.ds(base, CHUNK)], yv)
    pltpu.sync_copy(zs.at[pl.ds(base, CHUNK)], zv)
    pltpu.sync_copy(so.at[pl.ds(base, CHUNK)], sov)
    pltpu.sync_copy(batch2d.at[wid], idxv)
    pltpu.sync_copy(cxh, cxv)
    pltpu.sync_copy(cyh, cyv)
    pltpu.sync_copy(czh, czv)

    nseg = NSEG_W
    sbase = wid * nseg
    for k in range(nseg // LANES):
        stripe[pl.ds(k * LANES, LANES)] = jnp.zeros((LANES,), jnp.float32)
    pltpu.sync_copy(stripe, acc.at[pl.ds(sbase, nseg)])
    plsc.subcore_barrier()

    lane = lax.iota(jnp.int32, LANES)

    def chunk_build(j, carry):
        for k in range(8):
            i = j * 8 + k
            sl = pl.ds(i * LANES, LANES)
            ids = i * LANES + lane
            b16 = idxv[j, pl.ds(k * LANES, LANES)]
            dx = xv[sl] - plsc.load_gather(cxv, [b16])
            dy = yv[sl] - plsc.load_gather(cyv, [b16])
            dz = zv[sl] - plsc.load_gather(czv, [b16])
            sp = dx * dx + dy * dy + dz * dz
            valid = (base + ids) < N
            contrib[sl] = jnp.where(valid, sov[sl] * sp, 0.0)
        pltpu.async_copy(contrib.at[pl.ds(j * 128, 128)], acc.at[idxv.at[j]],
                         sem, add=True)
        return carry

    def chunk_drain(j, carry):
        pltpu.make_async_copy(contrib.at[pl.ds(j * 128, 128)],
                              acc.at[idxv.at[j]], sem).wait()
        return carry

    lax.fori_loop(0, NCH, chunk_build, 0)
    lax.fori_loop(0, NCH, chunk_drain, 0)
    plsc.subcore_barrier()
    pltpu.sync_copy(acc.at[pl.ds(sbase, nseg)], out.at[pl.ds(sbase, nseg)])


@functools.cache
def _extent_kernel(interpret: bool = False):
    return pl.kernel(
        _extent_body,
        out_type=jax.ShapeDtypeStruct((B,), jnp.float32),
        mesh=_mesh(),
        scratch_types=[
            pltpu.VMEM((CHUNK,), jnp.float32),
            pltpu.VMEM((CHUNK,), jnp.float32),
            pltpu.VMEM((CHUNK,), jnp.float32),
            pltpu.VMEM((CHUNK,), jnp.float32),
            pltpu.VMEM((NCH, 128), jnp.int32),
            pltpu.VMEM((B,), jnp.float32),
            pltpu.VMEM((B,), jnp.float32),
            pltpu.VMEM((B,), jnp.float32),
            pltpu.VMEM((CHUNK,), jnp.float32),
            pltpu.VMEM((NSEG_W,), jnp.float32),
            pltpu.VMEM_SHARED((B,), jnp.float32),
            pltpu.SemaphoreType.DMA,
        ],
        compiler_params=_SC_PARAMS,
        interpret=interpret,
    )


def _mlp_body(x_ref, w1_ref, b1_ref, w2t_ref, b2_ref, o_ref):
    h = jnp.dot(x_ref[...], w1_ref[...], preferred_element_type=jnp.float32)
    h = h + b1_ref[...]
    h = h * jax.nn.sigmoid(h)
    s = lax.dot_general(w2t_ref[...], h, (((1,), (1,)), ((), ())),
                        preferred_element_type=jnp.float32)
    o_ref[...] = (s + b2_ref[...])[None]


_MLP_BLOCK = 2048
_MLP_GRID = NP // _MLP_BLOCK


def _mlp(node_scalar, W1, b1, W2, b2):
    return pl.pallas_call(
        _mlp_body,
        grid=(_MLP_GRID,),
        in_specs=[
            pl.BlockSpec((_MLP_BLOCK, NODE_DIM), lambda i: (i, 0)),
            pl.BlockSpec((NODE_DIM, HIDDEN_DIM), lambda i: (0, 0)),
            pl.BlockSpec((HIDDEN_DIM,), lambda i: (0,)),
            pl.BlockSpec((1, HIDDEN_DIM), lambda i: (0, 0)),
            pl.BlockSpec((1,), lambda i: (0,)),
        ],
        out_specs=pl.BlockSpec((1, 1, _MLP_BLOCK), lambda i: (i, 0, 0)),
        out_shape=jax.ShapeDtypeStruct((_MLP_GRID, 1, _MLP_BLOCK),
                                       jnp.float32),
    )(node_scalar, W1, b1, W2.reshape(1, HIDDEN_DIM), b2)


def kernel(pos, node_scalar, W1, b1, W2, b2, masses_table, batch,
           atomic_numbers):
    pad = NP - N
    batch32 = jnp.concatenate(
        [batch.astype(jnp.int32), jnp.zeros((pad,), jnp.int32)])
    an32 = jnp.concatenate(
        [atomic_numbers.astype(jnp.int32),
         jnp.full((pad,), NSPECIES, jnp.int32)])
    table = jnp.concatenate(
        [masses_table, jnp.zeros((128 - NSPECIES,), jnp.float32)])
    zpad = jnp.zeros((pad,), jnp.float32)
    xs = jnp.concatenate([pos[:, 0], zpad])
    ys = jnp.concatenate([pos[:, 1], zpad])
    zs = jnp.concatenate([pos[:, 2], zpad])
    batch2d = batch32.reshape(NW, NCH, 128)

    cx, cy, cz = _centroid_kernel()(xs, ys, zs, an32, batch2d, table)
    so = _mlp(node_scalar, W1, b1, W2, b2).reshape(NP)
    out = _extent_kernel()(xs, ys, zs, so, batch2d, cx, cy, cz)
    return out.reshape(B, 1)


# speedup vs baseline: 19.7441x; 1.2059x over previous
import functools

import jax
import jax.numpy as jnp
from jax import lax
from jax.experimental import pallas as pl
from jax.experimental.pallas import tpu as pltpu
from jax.experimental.pallas import tpu_sc as plsc

N = 100000
B = 4096
NODE_DIM = 128
HIDDEN_DIM = 64
NSPECIES = 119

NW = 16
CHUNK = 6272
NCH = 49
NP = NW * CHUNK
LANES = 16
NSEG_W = B // NW


@functools.cache
def _mesh():
    return plsc.VectorSubcoreMesh(
        core_axis_name="c", subcore_axis_name="s", num_cores=1,
        num_subcores=NW)


_SC_PARAMS = pltpu.CompilerParams(needs_layout_passes=False)


def _centroid_body(xs, ys, zs, an, batch2d, table,
                   cx, cy, cz,
                   rx, ry, rz, rm, anv, idxv, tablev, stripe,
                   accx, accy, accz, accm, sem):
    wid = lax.axis_index("s")
    base = wid * CHUNK

    pltpu.sync_copy(xs.at[pl.ds(base, CHUNK)], rx)
    pltpu.sync_copy(ys.at[pl.ds(base, CHUNK)], ry)
    pltpu.sync_copy(zs.at[pl.ds(base, CHUNK)], rz)
    pltpu.sync_copy(an.at[pl.ds(base, CHUNK)], anv)
    pltpu.sync_copy(batch2d.at[wid], idxv)
    pltpu.sync_copy(table, tablev)

    for k in range(NSEG_W // LANES):
        stripe[pl.ds(k * LANES, LANES)] = jnp.zeros((LANES,), jnp.float32)
    sbase = wid * NSEG_W
    pltpu.sync_copy(stripe, accx.at[pl.ds(sbase, NSEG_W)])
    pltpu.sync_copy(stripe, accy.at[pl.ds(sbase, NSEG_W)])
    pltpu.sync_copy(stripe, accz.at[pl.ds(sbase, NSEG_W)])
    pltpu.sync_copy(stripe, accm.at[pl.ds(sbase, NSEG_W)])
    plsc.subcore_barrier()

    def chunk_build(j, carry):
        for k in range(8):
            i = j * 8 + k
            sl = pl.ds(i * LANES, LANES)
            m16 = plsc.load_gather(tablev, [anv[sl]])
            rm[sl] = m16
            rx[sl] = rx[sl] * m16
            ry[sl] = ry[sl] * m16
            rz[sl] = rz[sl] * m16
        rows = pl.ds(j * 128, 128)
        idx = idxv.at[j]
        pltpu.async_copy(rx.at[rows], accx.at[idx], sem, add=True)
        pltpu.async_copy(ry.at[rows], accy.at[idx], sem, add=True)
        pltpu.async_copy(rz.at[rows], accz.at[idx], sem, add=True)
        pltpu.async_copy(rm.at[rows], accm.at[idx], sem, add=True)
        return carry

    def chunk_drain(j, carry):
        rows = pl.ds(j * 128, 128)
        idx = idxv.at[j]
        pltpu.make_async_copy(rx.at[rows], accx.at[idx], sem).wait()
        pltpu.make_async_copy(ry.at[rows], accy.at[idx], sem).wait()
        pltpu.make_async_copy(rz.at[rows], accz.at[idx], sem).wait()
        pltpu.make_async_copy(rm.at[rows], accm.at[idx], sem).wait()
        return carry

    lax.fori_loop(0, NCH, chunk_build, 0)
    lax.fori_loop(0, NCH, chunk_drain, 0)
    plsc.subcore_barrier()

    num_x = rx.at[pl.ds(0, NSEG_W)]
    num_y = ry.at[pl.ds(0, NSEG_W)]
    num_z = rz.at[pl.ds(0, NSEG_W)]
    den = rm.at[pl.ds(0, NSEG_W)]
    pltpu.sync_copy(accx.at[pl.ds(sbase, NSEG_W)], num_x)
    pltpu.sync_copy(accy.at[pl.ds(sbase, NSEG_W)], num_y)
    pltpu.sync_copy(accz.at[pl.ds(sbase, NSEG_W)], num_z)
    pltpu.sync_copy(accm.at[pl.ds(sbase, NSEG_W)], den)

    def divide(k, carry):
        sl = pl.ds(k * LANES, LANES)
        inv = 1.0 / den[sl]
        num_x[sl] = num_x[sl] * inv
        num_y[sl] = num_y[sl] * inv
        num_z[sl] = num_z[sl] * inv
        return carry

    lax.fori_loop(0, NSEG_W // LANES, divide, 0)
    pltpu.sync_copy(num_x, cx.at[pl.ds(sbase, NSEG_W)])
    pltpu.sync_copy(num_y, cy.at[pl.ds(sbase, NSEG_W)])
    pltpu.sync_copy(num_z, cz.at[pl.ds(sbase, NSEG_W)])


@functools.cache
def _centroid_kernel(interpret: bool = False):
    return pl.kernel(
        _centroid_body,
        out_type=[jax.ShapeDtypeStruct((B,), jnp.float32)] * 3,
        mesh=_mesh(),
        scratch_types=[
            pltpu.VMEM((CHUNK,), jnp.float32),
            pltpu.VMEM((CHUNK,), jnp.float32),
            pltpu.VMEM((CHUNK,), jnp.float32),
            pltpu.VMEM((CHUNK,), jnp.float32),
            pltpu.VMEM((CHUNK,), jnp.int32),
            pltpu.VMEM((NCH, 128), jnp.int32),
            pltpu.VMEM((128,), jnp.float32),
            pltpu.VMEM((NSEG_W,), jnp.float32),
            pltpu.VMEM_SHARED((B,), jnp.float32),
            pltpu.VMEM_SHARED((B,), jnp.float32),
            pltpu.VMEM_SHARED((B,), jnp.float32),
            pltpu.VMEM_SHARED((B,), jnp.float32),
            pltpu.SemaphoreType.DMA,
        ],
        compiler_params=_SC_PARAMS,
        interpret=interpret,
    )


def _extent_body(xs, ys, zs, so, batch2d, cxh, cyh, czh,
                 out,
                 xv, yv, zv, sov, idxv, cxv, cyv, czv, contrib, stripe, acc,
                 sem):
    wid = lax.axis_index("s")
    base = wid * CHUNK

    pltpu.sync_copy(xs.at[pl.ds(base, CHUNK)], xv)
    pltpu.sync_copy(ys.at[pl.ds(base, CHUNK)], yv)
    pltpu.sync_copy(zs.at[pl.ds(base, CHUNK)], zv)
    pltpu.sync_copy(so.at[pl.ds(base, CHUNK)], sov)
    pltpu.sync_copy(batch2d.at[wid], idxv)
    pltpu.sync_copy(cxh, cxv)
    pltpu.sync_copy(cyh, cyv)
    pltpu.sync_copy(czh, czv)

    nseg = NSEG_W
    sbase = wid * nseg
    for k in range(nseg // LANES):
        stripe[pl.ds(k * LANES, LANES)] = jnp.zeros((LANES,), jnp.float32)
    pltpu.sync_copy(stripe, acc.at[pl.ds(sbase, nseg)])
    plsc.subcore_barrier()

    lane = lax.iota(jnp.int32, LANES)

    def chunk_build(j, carry):
        for k in range(8):
            i = j * 8 + k
            sl = pl.ds(i * LANES, LANES)
            ids = i * LANES + lane
            b16 = idxv[j, pl.ds(k * LANES, LANES)]
            dx = xv[sl] - plsc.load_gather(cxv, [b16])
            dy = yv[sl] - plsc.load_gather(cyv, [b16])
            dz = zv[sl] - plsc.load_gather(czv, [b16])
            sp = dx * dx + dy * dy + dz * dz
            valid = (base + ids) < N
            contrib[sl] = jnp.where(valid, sov[sl] * sp, 0.0)
        pltpu.async_copy(contrib.at[pl.ds(j * 128, 128)], acc.at[idxv.at[j]],
                         sem, add=True)
        return carry

    def chunk_drain(j, carry):
        pltpu.make_async_copy(contrib.at[pl.ds(j * 128, 128)],
                              acc.at[idxv.at[j]], sem).wait()
        return carry

    lax.fori_loop(0, NCH, chunk_build, 0)
    lax.fori_loop(0, NCH, chunk_drain, 0)
    plsc.subcore_barrier()
    pltpu.sync_copy(acc.at[pl.ds(sbase, nseg)], out.at[pl.ds(sbase, nseg)])


@functools.cache
def _extent_kernel(interpret: bool = False):
    return pl.kernel(
        _extent_body,
        out_type=jax.ShapeDtypeStruct((B,), jnp.float32),
        mesh=_mesh(),
        scratch_types=[
            pltpu.VMEM((CHUNK,), jnp.float32),
            pltpu.VMEM((CHUNK,), jnp.float32),
            pltpu.VMEM((CHUNK,), jnp.float32),
            pltpu.VMEM((CHUNK,), jnp.float32),
            pltpu.VMEM((NCH, 128), jnp.int32),
            pltpu.VMEM((B,), jnp.float32),
            pltpu.VMEM((B,), jnp.float32),
            pltpu.VMEM((B,), jnp.float32),
            pltpu.VMEM((CHUNK,), jnp.float32),
            pltpu.VMEM((NSEG_W,), jnp.float32),
            pltpu.VMEM_SHARED((B,), jnp.float32),
            pltpu.SemaphoreType.DMA,
        ],
        compiler_params=_SC_PARAMS,
        interpret=interpret,
    )


def _mlp_body(x_ref, w1_ref, b1_ref, w2t_ref, b2_ref, o_ref):
    h = jnp.dot(x_ref[...], w1_ref[...], preferred_element_type=jnp.float32)
    h = h + b1_ref[...]
    h = h * jax.nn.sigmoid(h)
    s = lax.dot_general(w2t_ref[...], h, (((1,), (1,)), ((), ())),
                        preferred_element_type=jnp.float32)
    o_ref[...] = (s + b2_ref[...])[None]


_MLP_BLOCK = 6272
_MLP_GRID = NP // _MLP_BLOCK


def _mlp(node_scalar, W1, b1, W2, b2):
    return pl.pallas_call(
        _mlp_body,
        grid=(_MLP_GRID,),
        in_specs=[
            pl.BlockSpec((_MLP_BLOCK, NODE_DIM), lambda i: (i, 0)),
            pl.BlockSpec((NODE_DIM, HIDDEN_DIM), lambda i: (0, 0)),
            pl.BlockSpec((HIDDEN_DIM,), lambda i: (0,)),
            pl.BlockSpec((1, HIDDEN_DIM), lambda i: (0, 0)),
            pl.BlockSpec((1,), lambda i: (0,)),
        ],
        out_specs=pl.BlockSpec((1, 1, _MLP_BLOCK), lambda i: (i, 0, 0)),
        out_shape=jax.ShapeDtypeStruct((_MLP_GRID, 1, _MLP_BLOCK),
                                       jnp.float32),
    )(node_scalar, W1, b1, W2.reshape(1, HIDDEN_DIM), b2)


def kernel(pos, node_scalar, W1, b1, W2, b2, masses_table, batch,
           atomic_numbers):
    pad = NP - N
    batch32 = jnp.concatenate(
        [batch.astype(jnp.int32), jnp.zeros((pad,), jnp.int32)])
    an32 = jnp.concatenate(
        [atomic_numbers.astype(jnp.int32),
         jnp.full((pad,), NSPECIES, jnp.int32)])
    table = jnp.concatenate(
        [masses_table, jnp.zeros((128 - NSPECIES,), jnp.float32)])
    zpad = jnp.zeros((pad,), jnp.float32)
    xs = jnp.concatenate([pos[:, 0], zpad])
    ys = jnp.concatenate([pos[:, 1], zpad])
    zs = jnp.concatenate([pos[:, 2], zpad])
    batch2d = batch32.reshape(NW, NCH, 128)

    cx, cy, cz = _centroid_kernel()(xs, ys, zs, an32, batch2d, table)
    so = _mlp(node_scalar, W1, b1, W2, b2).reshape(NP)
    out = _extent_kernel()(xs, ys, zs, so, batch2d, cx, cy, cz)
    return out.reshape(B, 1)


# speedup vs baseline: 20.3612x; 1.0313x over previous
import functools

import jax
import jax.numpy as jnp
from jax import lax
from jax.experimental import pallas as pl
from jax.experimental.pallas import tpu as pltpu
from jax.experimental.pallas import tpu_sc as plsc

N = 100000
B = 4096
NODE_DIM = 128
HIDDEN_DIM = 64
NSPECIES = 119

NW = 16
CHUNK = 6400
NCH = 50
NP = NW * CHUNK
CHUNK2 = 3200
NCH2 = 25
LANES = 16
NSEG_W = B // NW


@functools.cache
def _mesh():
    return plsc.VectorSubcoreMesh(
        core_axis_name="c", subcore_axis_name="s", num_cores=1,
        num_subcores=NW)


@functools.cache
def _mesh2():
    return plsc.VectorSubcoreMesh(
        core_axis_name="c", subcore_axis_name="s", num_cores=2,
        num_subcores=NW)


_SC_PARAMS = pltpu.CompilerParams(needs_layout_passes=False)


def _centroid_body(xs, ys, zs, an, batch2d, table,
                   cx, cy, cz,
                   rx, ry, rz, rm, anv, idxv, tablev, stripe,
                   accx, accy, accz, accm, sem):
    wid = lax.axis_index("s")
    base = wid * CHUNK

    pltpu.sync_copy(xs.at[pl.ds(base, CHUNK)], rx)
    pltpu.sync_copy(ys.at[pl.ds(base, CHUNK)], ry)
    pltpu.sync_copy(zs.at[pl.ds(base, CHUNK)], rz)
    pltpu.sync_copy(an.at[pl.ds(base, CHUNK)], anv)
    pltpu.sync_copy(batch2d.at[wid], idxv)
    pltpu.sync_copy(table, tablev)

    for k in range(NSEG_W // LANES):
        stripe[pl.ds(k * LANES, LANES)] = jnp.zeros((LANES,), jnp.float32)
    sbase = wid * NSEG_W
    pltpu.sync_copy(stripe, accx.at[pl.ds(sbase, NSEG_W)])
    pltpu.sync_copy(stripe, accy.at[pl.ds(sbase, NSEG_W)])
    pltpu.sync_copy(stripe, accz.at[pl.ds(sbase, NSEG_W)])
    pltpu.sync_copy(stripe, accm.at[pl.ds(sbase, NSEG_W)])
    plsc.subcore_barrier()

    def chunk_build(j, carry):
        for k in range(8):
            i = j * 8 + k
            sl = pl.ds(i * LANES, LANES)
            m16 = plsc.load_gather(tablev, [anv[sl]])
            rm[sl] = m16
            rx[sl] = rx[sl] * m16
            ry[sl] = ry[sl] * m16
            rz[sl] = rz[sl] * m16
        rows = pl.ds(j * 128, 128)
        idx = idxv.at[j]
        pltpu.async_copy(rx.at[rows], accx.at[idx], sem, add=True)
        pltpu.async_copy(ry.at[rows], accy.at[idx], sem, add=True)
        pltpu.async_copy(rz.at[rows], accz.at[idx], sem, add=True)
        pltpu.async_copy(rm.at[rows], accm.at[idx], sem, add=True)
        return carry

    def chunk_drain(j, carry):
        rows = pl.ds(j * 128, 128)
        idx = idxv.at[j]
        pltpu.make_async_copy(rx.at[rows], accx.at[idx], sem).wait()
        pltpu.make_async_copy(ry.at[rows], accy.at[idx], sem).wait()
        pltpu.make_async_copy(rz.at[rows], accz.at[idx], sem).wait()
        pltpu.make_async_copy(rm.at[rows], accm.at[idx], sem).wait()
        return carry

    lax.fori_loop(0, NCH, chunk_build, 0)
    lax.fori_loop(0, NCH, chunk_drain, 0)
    plsc.subcore_barrier()

    num_x = rx.at[pl.ds(0, NSEG_W)]
    num_y = ry.at[pl.ds(0, NSEG_W)]
    num_z = rz.at[pl.ds(0, NSEG_W)]
    den = rm.at[pl.ds(0, NSEG_W)]
    pltpu.sync_copy(accx.at[pl.ds(sbase, NSEG_W)], num_x)
    pltpu.sync_copy(accy.at[pl.ds(sbase, NSEG_W)], num_y)
    pltpu.sync_copy(accz.at[pl.ds(sbase, NSEG_W)], num_z)
    pltpu.sync_copy(accm.at[pl.ds(sbase, NSEG_W)], den)

    def divide(k, carry):
        sl = pl.ds(k * LANES, LANES)
        inv = 1.0 / den[sl]
        num_x[sl] = num_x[sl] * inv
        num_y[sl] = num_y[sl] * inv
        num_z[sl] = num_z[sl] * inv
        return carry

    lax.fori_loop(0, NSEG_W // LANES, divide, 0)
    pltpu.sync_copy(num_x, cx.at[pl.ds(sbase, NSEG_W)])
    pltpu.sync_copy(num_y, cy.at[pl.ds(sbase, NSEG_W)])
    pltpu.sync_copy(num_z, cz.at[pl.ds(sbase, NSEG_W)])


@functools.cache
def _centroid_kernel(interpret: bool = False):
    return pl.kernel(
        _centroid_body,
        out_type=[jax.ShapeDtypeStruct((B,), jnp.float32)] * 3,
        mesh=_mesh(),
        scratch_types=[
            pltpu.VMEM((CHUNK,), jnp.float32),
            pltpu.VMEM((CHUNK,), jnp.float32),
            pltpu.VMEM((CHUNK,), jnp.float32),
            pltpu.VMEM((CHUNK,), jnp.float32),
            pltpu.VMEM((CHUNK,), jnp.int32),
            pltpu.VMEM((NCH, 128), jnp.int32),
            pltpu.VMEM((128,), jnp.float32),
            pltpu.VMEM((NSEG_W,), jnp.float32),
            pltpu.VMEM_SHARED((B,), jnp.float32),
            pltpu.VMEM_SHARED((B,), jnp.float32),
            pltpu.VMEM_SHARED((B,), jnp.float32),
            pltpu.VMEM_SHARED((B,), jnp.float32),
            pltpu.SemaphoreType.DMA,
        ],
        compiler_params=_SC_PARAMS,
        interpret=interpret,
    )


def _extent_body(xs, ys, zs, so, batch2d, cxh, cyh, czh,
                 out,
                 xv, yv, zv, sov, idxv, cxv, cyv, czv, contrib, stripe, acc,
                 sem):
    cid = lax.axis_index("c")
    sid = lax.axis_index("s")
    wid = cid * NW + sid
    base = wid * CHUNK2

    pltpu.sync_copy(xs.at[pl.ds(base, CHUNK2)], xv)
    pltpu.sync_copy(ys.at[pl.ds(base, CHUNK2)], yv)
    pltpu.sync_copy(zs.at[pl.ds(base, CHUNK2)], zv)
    pltpu.sync_copy(so.at[pl.ds(base, CHUNK2)], sov)
    pltpu.sync_copy(batch2d.at[wid], idxv)
    pltpu.sync_copy(cxh, cxv)
    pltpu.sync_copy(cyh, cyv)
    pltpu.sync_copy(czh, czv)

    nseg = NSEG_W
    sbase = sid * nseg
    for k in range(nseg // LANES):
        stripe[pl.ds(k * LANES, LANES)] = jnp.zeros((LANES,), jnp.float32)
    pltpu.sync_copy(stripe, acc.at[pl.ds(sbase, nseg)])
    plsc.subcore_barrier()

    lane = lax.iota(jnp.int32, LANES)

    def chunk_build(j, carry):
        for k in range(8):
            i = j * 8 + k
            sl = pl.ds(i * LANES, LANES)
            ids = i * LANES + lane
            b16 = idxv[j, pl.ds(k * LANES, LANES)]
            dx = xv[sl] - plsc.load_gather(cxv, [b16])
            dy = yv[sl] - plsc.load_gather(cyv, [b16])
            dz = zv[sl] - plsc.load_gather(czv, [b16])
            sp = dx * dx + dy * dy + dz * dz
            valid = (base + ids) < N
            contrib[sl] = jnp.where(valid, sov[sl] * sp, 0.0)
        pltpu.async_copy(contrib.at[pl.ds(j * 128, 128)], acc.at[idxv.at[j]],
                         sem, add=True)
        return carry

    def chunk_drain(j, carry):
        pltpu.make_async_copy(contrib.at[pl.ds(j * 128, 128)],
                              acc.at[idxv.at[j]], sem).wait()
        return carry

    lax.fori_loop(0, NCH2, chunk_build, 0)
    lax.fori_loop(0, NCH2, chunk_drain, 0)
    plsc.subcore_barrier()
    pltpu.sync_copy(acc.at[pl.ds(sbase, nseg)],
                    out.at[cid, pl.ds(sbase, nseg)])


@functools.cache
def _extent_kernel(interpret: bool = False):
    return pl.kernel(
        _extent_body,
        out_type=jax.ShapeDtypeStruct((2, B), jnp.float32),
        mesh=_mesh2(),
        scratch_types=[
            pltpu.VMEM((CHUNK2,), jnp.float32),
            pltpu.VMEM((CHUNK2,), jnp.float32),
            pltpu.VMEM((CHUNK2,), jnp.float32),
            pltpu.VMEM((CHUNK2,), jnp.float32),
            pltpu.VMEM((NCH2, 128), jnp.int32),
            pltpu.VMEM((B,), jnp.float32),
            pltpu.VMEM((B,), jnp.float32),
            pltpu.VMEM((B,), jnp.float32),
            pltpu.VMEM((CHUNK2,), jnp.float32),
            pltpu.VMEM((NSEG_W,), jnp.float32),
            pltpu.VMEM_SHARED((B,), jnp.float32),
            pltpu.SemaphoreType.DMA,
        ],
        compiler_params=_SC_PARAMS,
        interpret=interpret,
    )


def _mlp_body(x_ref, w1_ref, b1_ref, w2t_ref, b2_ref, o_ref):
    h = jnp.dot(x_ref[...], w1_ref[...], preferred_element_type=jnp.float32)
    h = h + b1_ref[...]
    h = h * jax.nn.sigmoid(h)
    s = lax.dot_general(w2t_ref[...], h, (((1,), (1,)), ((), ())),
                        preferred_element_type=jnp.float32)
    o_ref[...] = (s + b2_ref[...])[None]


_MLP_BLOCK = 6400
_MLP_GRID = NP // _MLP_BLOCK


def _mlp(node_scalar, W1, b1, W2, b2):
    return pl.pallas_call(
        _mlp_body,
        grid=(_MLP_GRID,),
        in_specs=[
            pl.BlockSpec((_MLP_BLOCK, NODE_DIM), lambda i: (i, 0)),
            pl.BlockSpec((NODE_DIM, HIDDEN_DIM), lambda i: (0, 0)),
            pl.BlockSpec((HIDDEN_DIM,), lambda i: (0,)),
            pl.BlockSpec((1, HIDDEN_DIM), lambda i: (0, 0)),
            pl.BlockSpec((1,), lambda i: (0,)),
        ],
        out_specs=pl.BlockSpec((1, 1, _MLP_BLOCK), lambda i: (i, 0, 0)),
        out_shape=jax.ShapeDtypeStruct((_MLP_GRID, 1, _MLP_BLOCK),
                                       jnp.float32),
    )(node_scalar, W1, b1, W2.reshape(1, HIDDEN_DIM), b2)


def kernel(pos, node_scalar, W1, b1, W2, b2, masses_table, batch,
           atomic_numbers):
    pad = NP - N
    batch32 = jnp.concatenate(
        [batch.astype(jnp.int32), jnp.zeros((pad,), jnp.int32)])
    an32 = jnp.concatenate(
        [atomic_numbers.astype(jnp.int32),
         jnp.full((pad,), NSPECIES, jnp.int32)])
    table = jnp.concatenate(
        [masses_table, jnp.zeros((128 - NSPECIES,), jnp.float32)])
    zpad = jnp.zeros((pad,), jnp.float32)
    xs = jnp.concatenate([pos[:, 0], zpad])
    ys = jnp.concatenate([pos[:, 1], zpad])
    zs = jnp.concatenate([pos[:, 2], zpad])
    batch2d = batch32.reshape(NW, NCH, 128)
    batch2d2 = batch32.reshape(2 * NW, NCH2, 128)

    cx, cy, cz = _centroid_kernel()(xs, ys, zs, an32, batch2d, table)
    so = _mlp(node_scalar, W1, b1, W2, b2).reshape(NP)
    out2 = _extent_kernel()(xs, ys, zs, so, batch2d2, cx, cy, cz)
    return (out2[0] + out2[1]).reshape(B, 1)
